# baseline (speedup 1.0000x reference)
import functools

import jax
import jax.numpy as jnp
from jax.experimental import pallas as pl
from jax.experimental.pallas import tpu as pltpu
from jax.experimental.pallas import tpu_sc as plsc

N_NEI = 16
D_TAB = 80
EPS = 1e-5
TCOL = 512


def _table_body(f_ref, fdb_ref, p_ref, pdb_ref, tab_ref, rs_ref, *, nloc):
    j = pl.program_id(0)
    use_db = (j % nloc) >= (nloc // 2)
    fblk = jnp.where(use_db, fdb_ref[0], f_ref[0])
    pblk = jnp.where(use_db, pdb_ref[0], p_ref[0])
    ft = jnp.transpose(fblk)
    pp = jnp.concatenate(
        [pblk, jnp.zeros((13, pblk.shape[1]), jnp.float32)], axis=0)
    pt = jnp.transpose(pp)
    tab_ref[...] = jnp.concatenate([ft, pt], axis=1)
    rs_ref[0] = jnp.sum(fblk * fblk, axis=0)[None, :]


def _table(feat, feat_feadb, pcd, pcd_feadb):
    B, C, N = feat.shape
    M = N + feat_feadb.shape[2]
    nloc = M // TCOL
    half = nloc // 2

    def fmap(j):
        return (j // nloc, 0, jnp.minimum(j % nloc, half - 1))

    def dbmap(j):
        return (j // nloc, 0, jnp.maximum(j % nloc - half, 0))

    return pl.pallas_call(
        functools.partial(_table_body, nloc=nloc),
        grid=(B * nloc,),
        in_specs=[
            pl.BlockSpec((1, C, TCOL), fmap),
            pl.BlockSpec((1, C, TCOL), dbmap),
            pl.BlockSpec((1, 3, TCOL), fmap),
            pl.BlockSpec((1, 3, TCOL), dbmap),
        ],
        out_specs=[
            pl.BlockSpec((TCOL, D_TAB), lambda j: (j, 0)),
            pl.BlockSpec((1, 1, TCOL), lambda j: (j // nloc, 0, j % nloc)),
        ],
        out_shape=[
            jax.ShapeDtypeStruct((B * M, D_TAB), jnp.float32),
            jax.ShapeDtypeStruct((B, 1, M), jnp.float32),
        ],
    )(feat, feat_feadb, pcd, pcd_feadb)


def _knn_body(q_ref, t_ref, rs_ref, idx_ref, *, m_total):
    b = pl.program_id(0)
    q = q_ref[0]
    r = t_ref[:, 0:64]
    qs = jnp.sum(q * q, axis=0)[:, None]
    rs = rs_ref[0]
    d = qs + rs - 2.0 * jax.lax.dot_general(
        q, r, (((0,), (1,)), ((), ())), preferred_element_type=jnp.float32)
    nq = d.shape[0]
    nv = m_total // 128
    inf = jnp.float32(jnp.inf)
    d3 = jnp.reshape(d, (nq, nv, 128))
    cmin = jnp.min(d3, axis=1)
    liota = jax.lax.broadcasted_iota(jnp.int32, (nq, 128), 1)
    lsel = []
    for _ in range(N_NEI):
        lj = jnp.argmin(cmin, axis=1)[:, None]
        lsel.append(lj)
        cmin = jnp.where(liota == lj, inf, cmin)
    lanes = jnp.concatenate(lsel, axis=1)
    dparts = []
    gparts = []
    for c in range(nv):
        dparts.append(jnp.take_along_axis(d[:, c * 128:(c + 1) * 128],
                                          lanes, axis=1))
        gparts.append(lanes + c * 128)
    dc = jnp.concatenate(dparts, axis=1)
    gidx = jnp.concatenate(gparts, axis=1)
    siota = jax.lax.broadcasted_iota(jnp.int32, dc.shape, 1)
    big = jnp.int32(m_total)
    cols = []
    for _ in range(N_NEI):
        sj = jnp.argmin(dc, axis=1)[:, None]
        cols.append(jnp.min(jnp.where(siota == sj, gidx, big), axis=1,
                            keepdims=True))
        dc = jnp.where(siota == sj, inf, dc)
    idx_ref[0] = jnp.concatenate(cols, axis=1) + b * m_total


def _knn(feat, table, rsq):
    B, C, N = feat.shape
    M = table.shape[0] // B
    NQ = 256
    return pl.pallas_call(
        functools.partial(_knn_body, m_total=M),
        grid=(B, N // NQ),
        in_specs=[
            pl.BlockSpec((1, C, NQ), lambda b, i: (b, 0, i)),
            pl.BlockSpec((M, D_TAB), lambda b, i: (b, 0)),
            pl.BlockSpec((1, 1, M), lambda b, i: (b, 0, 0)),
        ],
        out_specs=pl.BlockSpec((1, NQ, N_NEI), lambda b, i: (b, i, 0)),
        out_shape=jax.ShapeDtypeStruct((B, N, N_NEI), jnp.int32),
    )(feat, table, rsq)


def _sc_gather(table, idx_flat):
    rows_total = idx_flat.shape[0]
    d = table.shape[1]
    info = plsc.get_sparse_core_info()
    nw = info.num_cores * info.num_subcores
    per_w = rows_total // nw
    ch = 128
    n_ch = per_w // ch
    mesh = plsc.VectorSubcoreMesh(core_axis_name="c", subcore_axis_name="s")

    @functools.partial(
        pl.kernel,
        out_type=jax.ShapeDtypeStruct((rows_total, d), jnp.float32),
        mesh=mesh,
        scratch_types=[
            pltpu.VMEM((ch,), jnp.int32),
            pltpu.VMEM((ch, d), jnp.float32),
            pltpu.SemaphoreType.DMA,
        ],
        compiler_params=pltpu.CompilerParams(use_tc_tiling_on_sc=False),
    )
    def k(table_hbm, idx_hbm, out_hbm, idx_v, rows_v, sem):
        wid = jax.lax.axis_index("s") * info.num_cores + jax.lax.axis_index("c")
        base = wid * per_w

        def body(c, carry):
            off = base + c * ch
            pltpu.sync_copy(idx_hbm.at[pl.ds(off, ch)], idx_v)
            pltpu.async_copy(table_hbm.at[idx_v], rows_v, sem).wait()
            pltpu.sync_copy(rows_v, out_hbm.at[pl.ds(off, ch)])
            return carry

        jax.lax.fori_loop(0, n_ch, body, 0)

    return k(table, idx_flat)


def _pcd16(p_ref):
    pblk = p_ref[0]
    pp = jnp.concatenate(
        [pblk, jnp.zeros((13, pblk.shape[1]), jnp.float32)], axis=0)
    return jnp.transpose(pp)


def _bn_stats(acc, nrow, w, b1, cnt):
    s = acc[nrow:nrow + 1, :]
    outer = acc[0:nrow, :]
    mean_x = s / cnt
    cov = outer / cnt - mean_x * jnp.reshape(mean_x, (nrow, 1))
    mean = jax.lax.dot_general(
        mean_x, w, (((1,), (1,)), ((), ())),
        preferred_element_type=jnp.float32) + b1
    wc = jax.lax.dot_general(
        w, cov, (((1,), (0,)), ((), ())), preferred_element_type=jnp.float32)
    var = jnp.reshape(jnp.sum(wc * w, axis=1), (1, w.shape[0]))
    return mean, var


def _compute_pe(g_ref, p_ref, acc1_ref, w1_ref, b1_ref, g1_ref, be1_ref,
                w2_ref, b2_ref, cnt):
    pb = p_ref.shape[2]
    rb = pb * N_NEI
    mean1, var1 = _bn_stats(acc1_ref[...], 16, w1_ref[...], b1_ref[...], cnt)
    gp = g_ref[:, 64:80]
    p = _pcd16(p_ref)
    prep = jnp.reshape(
        jnp.broadcast_to(p[:, None, :], (pb, N_NEI, 16)), (rb, 16))
    pr = prep - gp
    pe1 = jax.lax.dot_general(
        pr, w1_ref[...], (((1,), (1,)), ((), ())),
        preferred_element_type=jnp.float32) + b1_ref[...]
    xn = (pe1 - mean1) * jax.lax.rsqrt(var1 + EPS) * g1_ref[...] + be1_ref[...]
    z = jnp.maximum(xn, 0.0)
    return jax.lax.dot_general(
        z, w2_ref[...], (((1,), (1,)), ((), ())),
        preferred_element_type=jnp.float32) + b2_ref[...]


def _fused_body(g_ref, p_ref, f_ref,
                pw1_ref, pb1_ref, pg1_ref, pbe1_ref, pw2_ref, pb2_ref,
                aw1_ref, ab1_ref, ag1_ref, abe1_ref, aw2_ref, ab2_ref,
                out_ref, acc1_ref, acc2_ref, *, nblk, cnt):
    i = pl.program_id(0)
    ph = i // nblk
    pb = p_ref.shape[2]
    rb = pb * N_NEI

    @pl.when(i == 0)
    def _():
        acc1_ref[...] = jnp.zeros_like(acc1_ref)

    @pl.when(i == nblk)
    def _():
        acc2_ref[...] = jnp.zeros_like(acc2_ref)

    @pl.when(ph == 0)
    def _():
        gp = g_ref[:, 64:80]
        p = _pcd16(p_ref)
        prep = jnp.reshape(
            jnp.broadcast_to(p[:, None, :], (pb, N_NEI, 16)), (rb, 16))
        pr = prep - gp
        outer = jax.lax.dot_general(
            pr, pr, (((0,), (0,)), ((), ())),
            preferred_element_type=jnp.float32)
        acc1_ref[0:16, :] += outer
        acc1_ref[16:17, :] += jnp.sum(pr, axis=0)[None, :]

    @pl.when(ph == 1)
    def _():
        pe = _compute_pe(g_ref, p_ref, acc1_ref, pw1_ref, pb1_ref, pg1_ref,
                         pbe1_ref, pw2_ref, pb2_ref, cnt)
        f = jnp.transpose(f_ref[0])
        frep = jnp.reshape(
            jnp.broadcast_to(f[:, None, :], (pb, N_NEI, 64)), (rb, 64))
        x2 = (frep - g_ref[:, 0:64]) + pe
        outer2 = jax.lax.dot_general(
            x2, x2, (((0,), (0,)), ((), ())),
            preferred_element_type=jnp.float32)
        acc2_ref[0:64, :] += outer2
        acc2_ref[64:65, :] += jnp.sum(x2, axis=0)[None, :]

    @pl.when(ph == 2)
    def _():
        hid = aw1_ref.shape[0]
        pe = _compute_pe(g_ref, p_ref, acc1_ref, pw1_ref, pb1_ref, pg1_ref,
                         pbe1_ref, pw2_ref, pb2_ref, cnt)
        mean2, var2 = _bn_stats(acc2_ref[...], 64, aw1_ref[...],
                                ab1_ref[...], cnt)
        f = jnp.transpose(f_ref[0])
        frep = jnp.reshape(
            jnp.broadcast_to(f[:, None, :], (pb, N_NEI, 64)), (rb, 64))
        gfeat = g_ref[:, 0:64]
        x2 = (frep - gfeat) + pe
        ap = jax.lax.dot_general(
            x2, aw1_ref[...], (((1,), (1,)), ((), ())),
            preferred_element_type=jnp.float32) + ab1_ref[...]
        an = ((ap - mean2) * jax.lax.rsqrt(var2 + EPS) * ag1_ref[...]
              + abe1_ref[...])
        an = jnp.maximum(an, 0.0)
        wp = jax.lax.dot_general(
            an, aw2_ref[...], (((1,), (1,)), ((), ())),
            preferred_element_type=jnp.float32) + ab2_ref[...]
        wp3 = jnp.reshape(wp, (pb, N_NEI, 64))
        m = jnp.max(wp3, axis=1, keepdims=True)
        e = jnp.exp(wp3 - m)
        sm = e / jnp.sum(e, axis=1, keepdims=True)
        gf3 = jnp.reshape(gfeat + pe, (pb, N_NEI, 64))
        out = jnp.sum(sm * gf3, axis=1)
        out_ref[0] = jnp.transpose(out)


def _fused(g, pcd, feat, pw1, pb1, pg1, pbe1, pw2, pb2,
           aw1, ab1, ag1, abe1, aw2, ab2, rb):
    rows = g.shape[0]
    nblk = rows // rb
    pb = rb // N_NEI
    B, C, N = feat.shape
    npb = N // pb
    hid = aw1.shape[0]
    cnt = float(rows)

    def gmap(i):
        return (i % nblk, 0)

    def pmap(i):
        return ((i % nblk) // npb, 0, (i % nblk) % npb)

    def cmap(i):
        return (0, 0)

    out, _, _ = pl.pallas_call(
        functools.partial(_fused_body, nblk=nblk, cnt=cnt),
        grid=(3 * nblk,),
        in_specs=[
            pl.BlockSpec((rb, D_TAB), gmap),
            pl.BlockSpec((1, 3, pb), pmap),
            pl.BlockSpec((1, 64, pb), pmap),
            pl.BlockSpec((64, 16), cmap),
            pl.BlockSpec((1, 64), cmap),
            pl.BlockSpec((1, 64), cmap),
            pl.BlockSpec((1, 64), cmap),
            pl.BlockSpec((64, 64), cmap),
            pl.BlockSpec((1, 64), cmap),
            pl.BlockSpec((hid, 64), cmap),
            pl.BlockSpec((1, hid), cmap),
            pl.BlockSpec((1, hid), cmap),
            pl.BlockSpec((1, hid), cmap),
            pl.BlockSpec((64, hid), cmap),
            pl.BlockSpec((1, 64), cmap),
        ],
        out_specs=[
            pl.BlockSpec((1, C, pb), pmap),
            pl.BlockSpec((24, 16), cmap),
            pl.BlockSpec((72, 64), cmap),
        ],
        out_shape=[
            jax.ShapeDtypeStruct((B, C, N), jnp.float32),
            jax.ShapeDtypeStruct((24, 16), jnp.float32),
            jax.ShapeDtypeStruct((72, 64), jnp.float32),
        ],
    )(g, pcd, feat, pw1, pb1, pg1, pbe1, pw2, pb2,
      aw1, ab1, ag1, abe1, aw2, ab2)
    return out


def kernel(pcd, feat, pcd_feadb, feat_feadb,
           pos_w1, pos_b1, pos_g1, pos_be1, pos_w2, pos_b2,
           attn_w1, attn_b1, attn_g1, attn_be1, attn_w2, attn_b2):
    B, C, N = feat.shape
    rows = B * N * N_NEI
    RB = 2048

    table, rsq = _table(feat, feat_feadb, pcd, pcd_feadb)
    idx = _knn(feat, table, rsq)
    g = _sc_gather(table, idx.reshape(rows))

    w1p = jnp.concatenate(
        [pos_w1, jnp.zeros((pos_w1.shape[0], 13), jnp.float32)], axis=1)
    return _fused(g, pcd, feat, w1p,
                  pos_b1[None, :], pos_g1[None, :], pos_be1[None, :],
                  pos_w2, pos_b2[None, :],
                  attn_w1, attn_b1[None, :], attn_g1[None, :],
                  attn_be1[None, :], attn_w2, attn_b2[None, :], RB)


# speedup vs baseline: 1.4921x; 1.4921x over previous
import functools

import jax
import jax.numpy as jnp
from jax.experimental import pallas as pl
from jax.experimental.pallas import tpu as pltpu
from jax.experimental.pallas import tpu_sc as plsc

N_NEI = 16
D_TAB = 80
EPS = 1e-5
TCOL = 512


def _table_body(f_ref, fdb_ref, p_ref, pdb_ref, tab_ref, rs_ref, *, nloc):
    j = pl.program_id(0)
    use_db = (j % nloc) >= (nloc // 2)
    fblk = jnp.where(use_db, fdb_ref[0], f_ref[0])
    pblk = jnp.where(use_db, pdb_ref[0], p_ref[0])
    ft = jnp.transpose(fblk)
    pp = jnp.concatenate(
        [pblk, jnp.zeros((13, pblk.shape[1]), jnp.float32)], axis=0)
    pt = jnp.transpose(pp)
    tab_ref[...] = jnp.concatenate([ft, pt], axis=1)
    rs_ref[0] = jnp.sum(fblk * fblk, axis=0)[None, :]


def _table(feat, feat_feadb, pcd, pcd_feadb):
    B, C, N = feat.shape
    M = N + feat_feadb.shape[2]
    nloc = M // TCOL
    half = nloc // 2

    def fmap(j):
        return (j // nloc, 0, jnp.minimum(j % nloc, half - 1))

    def dbmap(j):
        return (j // nloc, 0, jnp.maximum(j % nloc - half, 0))

    return pl.pallas_call(
        functools.partial(_table_body, nloc=nloc),
        grid=(B * nloc,),
        in_specs=[
            pl.BlockSpec((1, C, TCOL), fmap),
            pl.BlockSpec((1, C, TCOL), dbmap),
            pl.BlockSpec((1, 3, TCOL), fmap),
            pl.BlockSpec((1, 3, TCOL), dbmap),
        ],
        out_specs=[
            pl.BlockSpec((TCOL, D_TAB), lambda j: (j, 0)),
            pl.BlockSpec((1, 1, TCOL), lambda j: (j // nloc, 0, j % nloc)),
        ],
        out_shape=[
            jax.ShapeDtypeStruct((B * M, D_TAB), jnp.float32),
            jax.ShapeDtypeStruct((B, 1, M), jnp.float32),
        ],
    )(feat, feat_feadb, pcd, pcd_feadb)


def _knn_body(q_ref, t_ref, rs_ref, idx_ref, *, m_total):
    b = pl.program_id(0)
    q = q_ref[0]
    r = t_ref[:, 0:64]
    qs = jnp.sum(q * q, axis=0)[:, None]
    rs = rs_ref[0]
    d = qs + rs - 2.0 * jax.lax.dot_general(
        q, r, (((0,), (1,)), ((), ())), preferred_element_type=jnp.float32)
    nq = d.shape[0]
    nv = m_total // 128
    inf = jnp.float32(jnp.inf)
    d3 = jnp.reshape(d, (nq, nv, 128))
    cmin = jnp.min(d3, axis=1)
    liota = jax.lax.broadcasted_iota(jnp.int32, (nq, 128), 1)
    lsel = []
    for _ in range(N_NEI):
        lj = jnp.argmin(cmin, axis=1)[:, None]
        lsel.append(lj)
        cmin = jnp.where(liota == lj, inf, cmin)
    lanes = jnp.concatenate(lsel, axis=1)
    dparts = []
    gparts = []
    for c in range(nv):
        dparts.append(jnp.take_along_axis(d[:, c * 128:(c + 1) * 128],
                                          lanes, axis=1))
        gparts.append(lanes + c * 128)
    dc = jnp.concatenate(dparts, axis=1)
    gidx = jnp.concatenate(gparts, axis=1)
    big = jnp.int32(m_total)
    cols = []
    for _ in range(N_NEI):
        mv = jnp.min(dc, axis=1, keepdims=True)
        jg = jnp.min(jnp.where(dc == mv, gidx, big), axis=1, keepdims=True)
        cols.append(jg)
        dc = jnp.where(gidx == jg, inf, dc)
    idx_ref[0] = jnp.concatenate(cols, axis=1) + b * m_total


def _knn(feat, table, rsq):
    B, C, N = feat.shape
    M = table.shape[0] // B
    NQ = 256
    return pl.pallas_call(
        functools.partial(_knn_body, m_total=M),
        grid=(B, N // NQ),
        in_specs=[
            pl.BlockSpec((1, C, NQ), lambda b, i: (b, 0, i)),
            pl.BlockSpec((M, D_TAB), lambda b, i: (b, 0)),
            pl.BlockSpec((1, 1, M), lambda b, i: (b, 0, 0)),
        ],
        out_specs=pl.BlockSpec((1, NQ, N_NEI), lambda b, i: (b, i, 0)),
        out_shape=jax.ShapeDtypeStruct((B, N, N_NEI), jnp.int32),
    )(feat, table, rsq)


def _sc_gather(table, idx_flat):
    rows_total = idx_flat.shape[0]
    d = table.shape[1]
    info = plsc.get_sparse_core_info()
    nw = info.num_cores * info.num_subcores
    per_w = rows_total // nw
    ch = 128
    n_ch = per_w // ch
    mesh = plsc.VectorSubcoreMesh(core_axis_name="c", subcore_axis_name="s")

    @functools.partial(
        pl.kernel,
        out_type=jax.ShapeDtypeStruct((rows_total, d), jnp.float32),
        mesh=mesh,
        scratch_types=[
            pltpu.VMEM((ch,), jnp.int32),
            pltpu.VMEM((ch, d), jnp.float32),
            pltpu.SemaphoreType.DMA,
        ],
        compiler_params=pltpu.CompilerParams(use_tc_tiling_on_sc=False),
    )
    def k(table_hbm, idx_hbm, out_hbm, idx_v, rows_v, sem):
        wid = jax.lax.axis_index("s") * info.num_cores + jax.lax.axis_index("c")
        base = wid * per_w

        def body(c, carry):
            off = base + c * ch
            pltpu.sync_copy(idx_hbm.at[pl.ds(off, ch)], idx_v)
            pltpu.async_copy(table_hbm.at[idx_v], rows_v, sem).wait()
            pltpu.sync_copy(rows_v, out_hbm.at[pl.ds(off, ch)])
            return carry

        jax.lax.fori_loop(0, n_ch, body, 0)

    return k(table, idx_flat)


def _pcd16(p_ref):
    pblk = p_ref[0]
    pp = jnp.concatenate(
        [pblk, jnp.zeros((13, pblk.shape[1]), jnp.float32)], axis=0)
    return jnp.transpose(pp)


def _bn_stats(acc, nrow, w, b1, cnt):
    s = acc[nrow:nrow + 1, :]
    outer = acc[0:nrow, :]
    mean_x = s / cnt
    cov = outer / cnt - mean_x * jnp.reshape(mean_x, (nrow, 1))
    mean = jax.lax.dot_general(
        mean_x, w, (((1,), (1,)), ((), ())),
        preferred_element_type=jnp.float32) + b1
    wc = jax.lax.dot_general(
        w, cov, (((1,), (0,)), ((), ())), preferred_element_type=jnp.float32)
    var = jnp.reshape(jnp.sum(wc * w, axis=1), (1, w.shape[0]))
    return mean, var


def _compute_pe(g_ref, p_ref, acc1_ref, w1_ref, b1_ref, g1_ref, be1_ref,
                w2_ref, b2_ref, cnt):
    pb = p_ref.shape[2]
    rb = pb * N_NEI
    mean1, var1 = _bn_stats(acc1_ref[...], 16, w1_ref[...], b1_ref[...], cnt)
    gp = g_ref[:, 64:80]
    p = _pcd16(p_ref)
    prep = jnp.reshape(
        jnp.broadcast_to(p[:, None, :], (pb, N_NEI, 16)), (rb, 16))
    pr = prep - gp
    pe1 = jax.lax.dot_general(
        pr, w1_ref[...], (((1,), (1,)), ((), ())),
        preferred_element_type=jnp.float32) + b1_ref[...]
    xn = (pe1 - mean1) * jax.lax.rsqrt(var1 + EPS) * g1_ref[...] + be1_ref[...]
    z = jnp.maximum(xn, 0.0)
    return jax.lax.dot_general(
        z, w2_ref[...], (((1,), (1,)), ((), ())),
        preferred_element_type=jnp.float32) + b2_ref[...]


def _fused_body(g_ref, p_ref, f_ref,
                pw1_ref, pb1_ref, pg1_ref, pbe1_ref, pw2_ref, pb2_ref,
                aw1_ref, ab1_ref, ag1_ref, abe1_ref, aw2_ref, ab2_ref,
                out_ref, acc1_ref, acc2_ref, *, nblk, cnt):
    i = pl.program_id(0)
    ph = i // nblk
    pb = p_ref.shape[2]
    rb = pb * N_NEI

    @pl.when(i == 0)
    def _():
        acc1_ref[...] = jnp.zeros_like(acc1_ref)

    @pl.when(i == nblk)
    def _():
        acc2_ref[...] = jnp.zeros_like(acc2_ref)

    @pl.when(ph == 0)
    def _():
        gp = g_ref[:, 64:80]
        p = _pcd16(p_ref)
        prep = jnp.reshape(
            jnp.broadcast_to(p[:, None, :], (pb, N_NEI, 16)), (rb, 16))
        pr = prep - gp
        outer = jax.lax.dot_general(
            pr, pr, (((0,), (0,)), ((), ())),
            preferred_element_type=jnp.float32)
        acc1_ref[0:16, :] += outer
        acc1_ref[16:17, :] += jnp.sum(pr, axis=0)[None, :]

    @pl.when(ph == 1)
    def _():
        pe = _compute_pe(g_ref, p_ref, acc1_ref, pw1_ref, pb1_ref, pg1_ref,
                         pbe1_ref, pw2_ref, pb2_ref, cnt)
        f = jnp.transpose(f_ref[0])
        frep = jnp.reshape(
            jnp.broadcast_to(f[:, None, :], (pb, N_NEI, 64)), (rb, 64))
        x2 = (frep - g_ref[:, 0:64]) + pe
        outer2 = jax.lax.dot_general(
            x2, x2, (((0,), (0,)), ((), ())),
            preferred_element_type=jnp.float32)
        acc2_ref[0:64, :] += outer2
        acc2_ref[64:65, :] += jnp.sum(x2, axis=0)[None, :]

    @pl.when(ph == 2)
    def _():
        hid = aw1_ref.shape[0]
        pe = _compute_pe(g_ref, p_ref, acc1_ref, pw1_ref, pb1_ref, pg1_ref,
                         pbe1_ref, pw2_ref, pb2_ref, cnt)
        mean2, var2 = _bn_stats(acc2_ref[...], 64, aw1_ref[...],
                                ab1_ref[...], cnt)
        f = jnp.transpose(f_ref[0])
        frep = jnp.reshape(
            jnp.broadcast_to(f[:, None, :], (pb, N_NEI, 64)), (rb, 64))
        gfeat = g_ref[:, 0:64]
        x2 = (frep - gfeat) + pe
        ap = jax.lax.dot_general(
            x2, aw1_ref[...], (((1,), (1,)), ((), ())),
            preferred_element_type=jnp.float32) + ab1_ref[...]
        an = ((ap - mean2) * jax.lax.rsqrt(var2 + EPS) * ag1_ref[...]
              + abe1_ref[...])
        an = jnp.maximum(an, 0.0)
        wp = jax.lax.dot_general(
            an, aw2_ref[...], (((1,), (1,)), ((), ())),
            preferred_element_type=jnp.float32) + ab2_ref[...]
        wp3 = jnp.reshape(wp, (pb, N_NEI, 64))
        m = jnp.max(wp3, axis=1, keepdims=True)
        e = jnp.exp(wp3 - m)
        sm = e / jnp.sum(e, axis=1, keepdims=True)
        gf3 = jnp.reshape(gfeat + pe, (pb, N_NEI, 64))
        out = jnp.sum(sm * gf3, axis=1)
        out_ref[0] = jnp.transpose(out)


def _fused(g, pcd, feat, pw1, pb1, pg1, pbe1, pw2, pb2,
           aw1, ab1, ag1, abe1, aw2, ab2, rb):
    rows = g.shape[0]
    nblk = rows // rb
    pb = rb // N_NEI
    B, C, N = feat.shape
    npb = N // pb
    hid = aw1.shape[0]
    cnt = float(rows)

    def gmap(i):
        return (i % nblk, 0)

    def pmap(i):
        return ((i % nblk) // npb, 0, (i % nblk) % npb)

    def cmap(i):
        return (0, 0)

    out, _, _ = pl.pallas_call(
        functools.partial(_fused_body, nblk=nblk, cnt=cnt),
        grid=(3 * nblk,),
        in_specs=[
            pl.BlockSpec((rb, D_TAB), gmap),
            pl.BlockSpec((1, 3, pb), pmap),
            pl.BlockSpec((1, 64, pb), pmap),
            pl.BlockSpec((64, 16), cmap),
            pl.BlockSpec((1, 64), cmap),
            pl.BlockSpec((1, 64), cmap),
            pl.BlockSpec((1, 64), cmap),
            pl.BlockSpec((64, 64), cmap),
            pl.BlockSpec((1, 64), cmap),
            pl.BlockSpec((hid, 64), cmap),
            pl.BlockSpec((1, hid), cmap),
            pl.BlockSpec((1, hid), cmap),
            pl.BlockSpec((1, hid), cmap),
            pl.BlockSpec((64, hid), cmap),
            pl.BlockSpec((1, 64), cmap),
        ],
        out_specs=[
            pl.BlockSpec((1, C, pb), pmap),
            pl.BlockSpec((24, 16), cmap),
            pl.BlockSpec((72, 64), cmap),
        ],
        out_shape=[
            jax.ShapeDtypeStruct((B, C, N), jnp.float32),
            jax.ShapeDtypeStruct((24, 16), jnp.float32),
            jax.ShapeDtypeStruct((72, 64), jnp.float32),
        ],
    )(g, pcd, feat, pw1, pb1, pg1, pbe1, pw2, pb2,
      aw1, ab1, ag1, abe1, aw2, ab2)
    return out


def kernel(pcd, feat, pcd_feadb, feat_feadb,
           pos_w1, pos_b1, pos_g1, pos_be1, pos_w2, pos_b2,
           attn_w1, attn_b1, attn_g1, attn_be1, attn_w2, attn_b2):
    B, C, N = feat.shape
    rows = B * N * N_NEI
    RB = 2048

    table, rsq = _table(feat, feat_feadb, pcd, pcd_feadb)
    idx = _knn(feat, table, rsq)
    g = _sc_gather(table, idx.reshape(rows))

    w1p = jnp.concatenate(
        [pos_w1, jnp.zeros((pos_w1.shape[0], 13), jnp.float32)], axis=1)
    return _fused(g, pcd, feat, w1p,
                  pos_b1[None, :], pos_g1[None, :], pos_be1[None, :],
                  pos_w2, pos_b2[None, :],
                  attn_w1, attn_b1[None, :], attn_g1[None, :],
                  attn_be1[None, :], attn_w2, attn_b2[None, :], RB)


# speedup vs baseline: 1.5083x; 1.0109x over previous
import functools

import jax
import jax.numpy as jnp
from jax.experimental import pallas as pl
from jax.experimental.pallas import tpu as pltpu
from jax.experimental.pallas import tpu_sc as plsc

N_NEI = 16
D_TAB = 80
EPS = 1e-5
TCOL = 512


def _table_body(f_ref, fdb_ref, p_ref, pdb_ref, tab_ref, rs_ref, *, nloc):
    j = pl.program_id(0)
    use_db = (j % nloc) >= (nloc // 2)
    fblk = jnp.where(use_db, fdb_ref[0], f_ref[0])
    pblk = jnp.where(use_db, pdb_ref[0], p_ref[0])
    ft = jnp.transpose(fblk)
    pp = jnp.concatenate(
        [pblk, jnp.zeros((13, pblk.shape[1]), jnp.float32)], axis=0)
    pt = jnp.transpose(pp)
    tab_ref[...] = jnp.concatenate([ft, pt], axis=1)
    rs_ref[0] = jnp.sum(fblk * fblk, axis=0)[None, :]


def _table(feat, feat_feadb, pcd, pcd_feadb):
    B, C, N = feat.shape
    M = N + feat_feadb.shape[2]
    nloc = M // TCOL
    half = nloc // 2

    def fmap(j):
        return (j // nloc, 0, jnp.minimum(j % nloc, half - 1))

    def dbmap(j):
        return (j // nloc, 0, jnp.maximum(j % nloc - half, 0))

    return pl.pallas_call(
        functools.partial(_table_body, nloc=nloc),
        grid=(B * nloc,),
        in_specs=[
            pl.BlockSpec((1, C, TCOL), fmap),
            pl.BlockSpec((1, C, TCOL), dbmap),
            pl.BlockSpec((1, 3, TCOL), fmap),
            pl.BlockSpec((1, 3, TCOL), dbmap),
        ],
        out_specs=[
            pl.BlockSpec((TCOL, D_TAB), lambda j: (j, 0)),
            pl.BlockSpec((1, 1, TCOL), lambda j: (j // nloc, 0, j % nloc)),
        ],
        out_shape=[
            jax.ShapeDtypeStruct((B * M, D_TAB), jnp.float32),
            jax.ShapeDtypeStruct((B, 1, M), jnp.float32),
        ],
    )(feat, feat_feadb, pcd, pcd_feadb)


def _knn_body(q_ref, t_ref, rs_ref, idx_ref, *, m_total):
    b = pl.program_id(0)
    q = q_ref[0]
    r = t_ref[:, 0:64]
    qs = jnp.sum(q * q, axis=0)[:, None]
    rs = rs_ref[0]
    d = qs + rs - 2.0 * jax.lax.dot_general(
        q, r, (((0,), (1,)), ((), ())), preferred_element_type=jnp.float32)
    nq = d.shape[0]
    nv = m_total // 128
    inf = jnp.float32(jnp.inf)
    d3 = jnp.reshape(d, (nq, nv, 128))
    cmin = jnp.min(d3, axis=1)
    liota = jax.lax.broadcasted_iota(jnp.int32, (nq, 128), 1)
    lsel = []
    for _ in range(N_NEI):
        lj = jnp.argmin(cmin, axis=1)[:, None]
        lsel.append(lj)
        cmin = jnp.where(liota == lj, inf, cmin)
    lanes = jnp.concatenate(lsel, axis=1)
    dparts = []
    gparts = []
    for c in range(nv):
        dparts.append(jnp.take_along_axis(d[:, c * 128:(c + 1) * 128],
                                          lanes, axis=1))
        gparts.append(lanes + c * 128)
    dc = jnp.concatenate(dparts, axis=1)
    gidx = jnp.concatenate(gparts, axis=1)
    big = jnp.int32(m_total)
    cols = []
    for _ in range(N_NEI):
        mv = jnp.min(dc, axis=1, keepdims=True)
        jg = jnp.min(jnp.where(dc == mv, gidx, big), axis=1, keepdims=True)
        cols.append(jg)
        dc = jnp.where(gidx == jg, inf, dc)
    idx_ref[0] = jnp.concatenate(cols, axis=1) + b * m_total


def _knn(feat, table, rsq):
    B, C, N = feat.shape
    M = table.shape[0] // B
    NQ = 256
    return pl.pallas_call(
        functools.partial(_knn_body, m_total=M),
        grid=(B, N // NQ),
        in_specs=[
            pl.BlockSpec((1, C, NQ), lambda b, i: (b, 0, i)),
            pl.BlockSpec((M, D_TAB), lambda b, i: (b, 0)),
            pl.BlockSpec((1, 1, M), lambda b, i: (b, 0, 0)),
        ],
        out_specs=pl.BlockSpec((1, NQ, N_NEI), lambda b, i: (b, i, 0)),
        out_shape=jax.ShapeDtypeStruct((B, N, N_NEI), jnp.int32),
    )(feat, table, rsq)


def _sc_gather(table, idx_flat):
    rows_total = idx_flat.shape[0]
    d = table.shape[1]
    info = plsc.get_sparse_core_info()
    nw = info.num_cores * info.num_subcores
    per_w = rows_total // nw
    ch = 128
    n_ch = per_w // ch
    mesh = plsc.VectorSubcoreMesh(core_axis_name="c", subcore_axis_name="s")

    @functools.partial(
        pl.kernel,
        out_type=jax.ShapeDtypeStruct((rows_total, d), jnp.float32),
        mesh=mesh,
        scratch_types=[
            pltpu.VMEM((2, ch), jnp.int32),
            pltpu.VMEM((2, ch, d), jnp.float32),
            pltpu.SemaphoreType.DMA,
            pltpu.SemaphoreType.DMA,
        ],
        compiler_params=pltpu.CompilerParams(use_tc_tiling_on_sc=False),
    )
    def k(table_hbm, idx_hbm, out_hbm, idx_v, rows_v, gsem, osem):
        wid = jax.lax.axis_index("s") * info.num_cores + jax.lax.axis_index("c")
        base = wid * per_w

        def gwait(buf):
            pltpu.make_async_copy(
                table_hbm.at[idx_v.at[buf]], rows_v.at[buf], gsem).wait()

        def owait(buf):
            pltpu.make_async_copy(
                rows_v.at[buf], out_hbm.at[pl.ds(base, ch)], osem).wait()

        pltpu.sync_copy(idx_hbm.at[pl.ds(base, ch)], idx_v.at[0])
        pltpu.async_copy(table_hbm.at[idx_v.at[0]], rows_v.at[0], gsem)

        def body(c0, carry):
            for s in range(2):
                c = c0 + s
                cur = s
                nxt = 1 - s

                @pl.when(c + 1 < n_ch)
                def _():
                    pltpu.sync_copy(
                        idx_hbm.at[pl.ds(base + (c + 1) * ch, ch)],
                        idx_v.at[nxt])

                    @pl.when(c >= 1)
                    def _():
                        owait(nxt)

                    pltpu.async_copy(
                        table_hbm.at[idx_v.at[nxt]], rows_v.at[nxt], gsem)

                gwait(cur)
                pltpu.async_copy(
                    rows_v.at[cur], out_hbm.at[pl.ds(base + c * ch, ch)],
                    osem)
            return carry

        jax.lax.fori_loop(0, n_ch // 2, lambda j, car: body(j * 2, car), 0)
        owait(0)
        owait(1)

    return k(table, idx_flat)


def _pcd16(p_ref):
    pblk = p_ref[0]
    pp = jnp.concatenate(
        [pblk, jnp.zeros((13, pblk.shape[1]), jnp.float32)], axis=0)
    return jnp.transpose(pp)


def _bn_stats(acc, nrow, w, b1, cnt):
    s = acc[nrow:nrow + 1, :]
    outer = acc[0:nrow, :]
    mean_x = s / cnt
    cov = outer / cnt - mean_x * jnp.reshape(mean_x, (nrow, 1))
    mean = jax.lax.dot_general(
        mean_x, w, (((1,), (1,)), ((), ())),
        preferred_element_type=jnp.float32) + b1
    wc = jax.lax.dot_general(
        w, cov, (((1,), (0,)), ((), ())), preferred_element_type=jnp.float32)
    var = jnp.reshape(jnp.sum(wc * w, axis=1), (1, w.shape[0]))
    return mean, var


def _compute_pe(g_ref, p_ref, acc1_ref, w1_ref, b1_ref, g1_ref, be1_ref,
                w2_ref, b2_ref, cnt):
    pb = p_ref.shape[2]
    rb = pb * N_NEI
    mean1, var1 = _bn_stats(acc1_ref[...], 16, w1_ref[...], b1_ref[...], cnt)
    gp = g_ref[:, 64:80]
    p = _pcd16(p_ref)
    prep = jnp.reshape(
        jnp.broadcast_to(p[:, None, :], (pb, N_NEI, 16)), (rb, 16))
    pr = prep - gp
    pe1 = jax.lax.dot_general(
        pr, w1_ref[...], (((1,), (1,)), ((), ())),
        preferred_element_type=jnp.float32) + b1_ref[...]
    xn = (pe1 - mean1) * jax.lax.rsqrt(var1 + EPS) * g1_ref[...] + be1_ref[...]
    z = jnp.maximum(xn, 0.0)
    return jax.lax.dot_general(
        z, w2_ref[...], (((1,), (1,)), ((), ())),
        preferred_element_type=jnp.float32) + b2_ref[...]


def _fused_body(g_ref, p_ref, f_ref,
                pw1_ref, pb1_ref, pg1_ref, pbe1_ref, pw2_ref, pb2_ref,
                aw1_ref, ab1_ref, ag1_ref, abe1_ref, aw2_ref, ab2_ref,
                out_ref, acc1_ref, acc2_ref, *, nblk, cnt):
    i = pl.program_id(0)
    ph = i // nblk
    pb = p_ref.shape[2]
    rb = pb * N_NEI

    @pl.when(i == 0)
    def _():
        acc1_ref[...] = jnp.zeros_like(acc1_ref)

    @pl.when(i == nblk)
    def _():
        acc2_ref[...] = jnp.zeros_like(acc2_ref)

    @pl.when(ph == 0)
    def _():
        gp = g_ref[:, 64:80]
        p = _pcd16(p_ref)
        prep = jnp.reshape(
            jnp.broadcast_to(p[:, None, :], (pb, N_NEI, 16)), (rb, 16))
        pr = prep - gp
        outer = jax.lax.dot_general(
            pr, pr, (((0,), (0,)), ((), ())),
            preferred_element_type=jnp.float32)
        acc1_ref[0:16, :] += outer
        acc1_ref[16:17, :] += jnp.sum(pr, axis=0)[None, :]

    @pl.when(ph == 1)
    def _():
        pe = _compute_pe(g_ref, p_ref, acc1_ref, pw1_ref, pb1_ref, pg1_ref,
                         pbe1_ref, pw2_ref, pb2_ref, cnt)
        f = jnp.transpose(f_ref[0])
        frep = jnp.reshape(
            jnp.broadcast_to(f[:, None, :], (pb, N_NEI, 64)), (rb, 64))
        x2 = (frep - g_ref[:, 0:64]) + pe
        outer2 = jax.lax.dot_general(
            x2, x2, (((0,), (0,)), ((), ())),
            preferred_element_type=jnp.float32)
        acc2_ref[0:64, :] += outer2
        acc2_ref[64:65, :] += jnp.sum(x2, axis=0)[None, :]

    @pl.when(ph == 2)
    def _():
        hid = aw1_ref.shape[0]
        pe = _compute_pe(g_ref, p_ref, acc1_ref, pw1_ref, pb1_ref, pg1_ref,
                         pbe1_ref, pw2_ref, pb2_ref, cnt)
        mean2, var2 = _bn_stats(acc2_ref[...], 64, aw1_ref[...],
                                ab1_ref[...], cnt)
        f = jnp.transpose(f_ref[0])
        frep = jnp.reshape(
            jnp.broadcast_to(f[:, None, :], (pb, N_NEI, 64)), (rb, 64))
        gfeat = g_ref[:, 0:64]
        x2 = (frep - gfeat) + pe
        ap = jax.lax.dot_general(
            x2, aw1_ref[...], (((1,), (1,)), ((), ())),
            preferred_element_type=jnp.float32) + ab1_ref[...]
        an = ((ap - mean2) * jax.lax.rsqrt(var2 + EPS) * ag1_ref[...]
              + abe1_ref[...])
        an = jnp.maximum(an, 0.0)
        wp = jax.lax.dot_general(
            an, aw2_ref[...], (((1,), (1,)), ((), ())),
            preferred_element_type=jnp.float32) + ab2_ref[...]
        wp3 = jnp.reshape(wp, (pb, N_NEI, 64))
        m = jnp.max(wp3, axis=1, keepdims=True)
        e = jnp.exp(wp3 - m)
        sm = e / jnp.sum(e, axis=1, keepdims=True)
        gf3 = jnp.reshape(gfeat + pe, (pb, N_NEI, 64))
        out = jnp.sum(sm * gf3, axis=1)
        out_ref[0] = jnp.transpose(out)


def _fused(g, pcd, feat, pw1, pb1, pg1, pbe1, pw2, pb2,
           aw1, ab1, ag1, abe1, aw2, ab2, rb):
    rows = g.shape[0]
    nblk = rows // rb
    pb = rb // N_NEI
    B, C, N = feat.shape
    npb = N // pb
    hid = aw1.shape[0]
    cnt = float(rows)

    def gmap(i):
        return (i % nblk, 0)

    def pmap(i):
        return ((i % nblk) // npb, 0, (i % nblk) % npb)

    def cmap(i):
        return (0, 0)

    out, _, _ = pl.pallas_call(
        functools.partial(_fused_body, nblk=nblk, cnt=cnt),
        grid=(3 * nblk,),
        in_specs=[
            pl.BlockSpec((rb, D_TAB), gmap),
            pl.BlockSpec((1, 3, pb), pmap),
            pl.BlockSpec((1, 64, pb), pmap),
            pl.BlockSpec((64, 16), cmap),
            pl.BlockSpec((1, 64), cmap),
            pl.BlockSpec((1, 64), cmap),
            pl.BlockSpec((1, 64), cmap),
            pl.BlockSpec((64, 64), cmap),
            pl.BlockSpec((1, 64), cmap),
            pl.BlockSpec((hid, 64), cmap),
            pl.BlockSpec((1, hid), cmap),
            pl.BlockSpec((1, hid), cmap),
            pl.BlockSpec((1, hid), cmap),
            pl.BlockSpec((64, hid), cmap),
            pl.BlockSpec((1, 64), cmap),
        ],
        out_specs=[
            pl.BlockSpec((1, C, pb), pmap),
            pl.BlockSpec((24, 16), cmap),
            pl.BlockSpec((72, 64), cmap),
        ],
        out_shape=[
            jax.ShapeDtypeStruct((B, C, N), jnp.float32),
            jax.ShapeDtypeStruct((24, 16), jnp.float32),
            jax.ShapeDtypeStruct((72, 64), jnp.float32),
        ],
    )(g, pcd, feat, pw1, pb1, pg1, pbe1, pw2, pb2,
      aw1, ab1, ag1, abe1, aw2, ab2)
    return out


def kernel(pcd, feat, pcd_feadb, feat_feadb,
           pos_w1, pos_b1, pos_g1, pos_be1, pos_w2, pos_b2,
           attn_w1, attn_b1, attn_g1, attn_be1, attn_w2, attn_b2):
    B, C, N = feat.shape
    rows = B * N * N_NEI
    RB = 2048

    table, rsq = _table(feat, feat_feadb, pcd, pcd_feadb)
    idx = _knn(feat, table, rsq)
    g = _sc_gather(table, idx.reshape(rows))

    w1p = jnp.concatenate(
        [pos_w1, jnp.zeros((pos_w1.shape[0], 13), jnp.float32)], axis=1)
    return _fused(g, pcd, feat, w1p,
                  pos_b1[None, :], pos_g1[None, :], pos_be1[None, :],
                  pos_w2, pos_b2[None, :],
                  attn_w1, attn_b1[None, :], attn_g1[None, :],
                  attn_be1[None, :], attn_w2, attn_b2[None, :], RB)


# speedup vs baseline: 1.6861x; 1.1179x over previous
import functools

import jax
import jax.numpy as jnp
from jax.experimental import pallas as pl
from jax.experimental.pallas import tpu as pltpu
from jax.experimental.pallas import tpu_sc as plsc

N_NEI = 16
D_TAB = 80
EPS = 1e-5
TCOL = 512


def _table_body(f_ref, fdb_ref, p_ref, pdb_ref, tab_ref, rs_ref, *, nloc):
    j = pl.program_id(0)
    use_db = (j % nloc) >= (nloc // 2)
    fblk = jnp.where(use_db, fdb_ref[0], f_ref[0])
    pblk = jnp.where(use_db, pdb_ref[0], p_ref[0])
    ft = jnp.transpose(fblk)
    pp = jnp.concatenate(
        [pblk, jnp.zeros((13, pblk.shape[1]), jnp.float32)], axis=0)
    pt = jnp.transpose(pp)
    tab_ref[...] = jnp.concatenate([ft, pt], axis=1)
    rs_ref[0] = jnp.sum(fblk * fblk, axis=0)[None, :]


def _table(feat, feat_feadb, pcd, pcd_feadb):
    B, C, N = feat.shape
    M = N + feat_feadb.shape[2]
    nloc = M // TCOL
    half = nloc // 2

    def fmap(j):
        return (j // nloc, 0, jnp.minimum(j % nloc, half - 1))

    def dbmap(j):
        return (j // nloc, 0, jnp.maximum(j % nloc - half, 0))

    return pl.pallas_call(
        functools.partial(_table_body, nloc=nloc),
        grid=(B * nloc,),
        in_specs=[
            pl.BlockSpec((1, C, TCOL), fmap),
            pl.BlockSpec((1, C, TCOL), dbmap),
            pl.BlockSpec((1, 3, TCOL), fmap),
            pl.BlockSpec((1, 3, TCOL), dbmap),
        ],
        out_specs=[
            pl.BlockSpec((TCOL, D_TAB), lambda j: (j, 0)),
            pl.BlockSpec((1, 1, TCOL), lambda j: (j // nloc, 0, j % nloc)),
        ],
        out_shape=[
            jax.ShapeDtypeStruct((B * M, D_TAB), jnp.float32),
            jax.ShapeDtypeStruct((B, 1, M), jnp.float32),
        ],
    )(feat, feat_feadb, pcd, pcd_feadb)


def _knn_body(q_ref, t_ref, rs_ref, idx_ref, *, m_total):
    b = pl.program_id(0)
    q = q_ref[0]
    r = t_ref[:, 0:64]
    qs = jnp.sum(q * q, axis=0)[:, None]
    rs = rs_ref[0]
    d = qs + rs - 2.0 * jax.lax.dot_general(
        q, r, (((0,), (1,)), ((), ())), preferred_element_type=jnp.float32)
    nq = d.shape[0]
    nv = m_total // 128
    inf = jnp.float32(jnp.inf)
    d3 = jnp.reshape(d, (nq, nv, 128))
    cmin = jnp.min(d3, axis=1)
    liota = jax.lax.broadcasted_iota(jnp.int32, (nq, 128), 1)
    lsel = []
    for _ in range(N_NEI):
        lj = jnp.argmin(cmin, axis=1)[:, None]
        lsel.append(lj)
        cmin = jnp.where(liota == lj, inf, cmin)
    lanes = jnp.concatenate(lsel, axis=1)
    dparts = []
    gparts = []
    for c in range(nv):
        dparts.append(jnp.take_along_axis(d[:, c * 128:(c + 1) * 128],
                                          lanes, axis=1))
        gparts.append(lanes + c * 128)
    dc = jnp.concatenate(dparts, axis=1)
    gidx = jnp.concatenate(gparts, axis=1)
    big = jnp.int32(m_total)
    cols = []
    for _ in range(N_NEI):
        mv = jnp.min(dc, axis=1, keepdims=True)
        jg = jnp.min(jnp.where(dc == mv, gidx, big), axis=1, keepdims=True)
        cols.append(jg)
        dc = jnp.where(gidx == jg, inf, dc)
    idx_ref[0] = jnp.concatenate(cols, axis=1) + b * m_total


def _knn(feat, table, rsq):
    B, C, N = feat.shape
    M = table.shape[0] // B
    NQ = 512
    return pl.pallas_call(
        functools.partial(_knn_body, m_total=M),
        grid=(B, N // NQ),
        in_specs=[
            pl.BlockSpec((1, C, NQ), lambda b, i: (b, 0, i)),
            pl.BlockSpec((M, D_TAB), lambda b, i: (b, 0)),
            pl.BlockSpec((1, 1, M), lambda b, i: (b, 0, 0)),
        ],
        out_specs=pl.BlockSpec((1, NQ, N_NEI), lambda b, i: (b, i, 0)),
        out_shape=jax.ShapeDtypeStruct((B, N, N_NEI), jnp.int32),
    )(feat, table, rsq)


def _sc_gather(table, idx_flat):
    rows_total = idx_flat.shape[0]
    d = table.shape[1]
    info = plsc.get_sparse_core_info()
    nw = info.num_cores * info.num_subcores
    per_w = rows_total // nw
    ch = 128
    n_ch = per_w // ch
    mesh = plsc.VectorSubcoreMesh(core_axis_name="c", subcore_axis_name="s")

    @functools.partial(
        pl.kernel,
        out_type=jax.ShapeDtypeStruct((rows_total, d), jnp.float32),
        mesh=mesh,
        scratch_types=[
            pltpu.VMEM((2, ch), jnp.int32),
            pltpu.VMEM((2, ch, d), jnp.float32),
            pltpu.SemaphoreType.DMA,
            pltpu.SemaphoreType.DMA,
        ],
        compiler_params=pltpu.CompilerParams(use_tc_tiling_on_sc=False),
    )
    def k(table_hbm, idx_hbm, out_hbm, idx_v, rows_v, gsem, osem):
        wid = jax.lax.axis_index("s") * info.num_cores + jax.lax.axis_index("c")
        base = wid * per_w

        def gwait(buf):
            pltpu.make_async_copy(
                table_hbm.at[idx_v.at[buf]], rows_v.at[buf], gsem).wait()

        def owait(buf):
            pltpu.make_async_copy(
                rows_v.at[buf], out_hbm.at[pl.ds(base, ch)], osem).wait()

        pltpu.sync_copy(idx_hbm.at[pl.ds(base, ch)], idx_v.at[0])
        pltpu.async_copy(table_hbm.at[idx_v.at[0]], rows_v.at[0], gsem)

        def body(c0, carry):
            for s in range(2):
                c = c0 + s
                cur = s
                nxt = 1 - s

                @pl.when(c + 1 < n_ch)
                def _():
                    pltpu.sync_copy(
                        idx_hbm.at[pl.ds(base + (c + 1) * ch, ch)],
                        idx_v.at[nxt])

                    @pl.when(c >= 1)
                    def _():
                        owait(nxt)

                    pltpu.async_copy(
                        table_hbm.at[idx_v.at[nxt]], rows_v.at[nxt], gsem)

                gwait(cur)
                pltpu.async_copy(
                    rows_v.at[cur], out_hbm.at[pl.ds(base + c * ch, ch)],
                    osem)
            return carry

        jax.lax.fori_loop(0, n_ch // 2, lambda j, car: body(j * 2, car), 0)
        owait(0)
        owait(1)

    return k(table, idx_flat)


def _pcd16(p_ref):
    pblk = p_ref[0]
    pp = jnp.concatenate(
        [pblk, jnp.zeros((13, pblk.shape[1]), jnp.float32)], axis=0)
    return jnp.transpose(pp)


def _bn_stats(acc, nrow, w, b1, cnt):
    s = acc[nrow:nrow + 1, :]
    outer = acc[0:nrow, :]
    mean_x = s / cnt
    cov = outer / cnt - mean_x * jnp.reshape(mean_x, (nrow, 1))
    mean = jax.lax.dot_general(
        mean_x, w, (((1,), (1,)), ((), ())),
        preferred_element_type=jnp.float32) + b1
    wc = jax.lax.dot_general(
        w, cov, (((1,), (0,)), ((), ())), preferred_element_type=jnp.float32)
    var = jnp.reshape(jnp.sum(wc * w, axis=1), (1, w.shape[0]))
    return mean, var


def _compute_pe(g_ref, p_ref, acc1_ref, w1_ref, b1_ref, g1_ref, be1_ref,
                w2_ref, b2_ref, cnt):
    pb = p_ref.shape[2]
    rb = pb * N_NEI
    mean1, var1 = _bn_stats(acc1_ref[...], 16, w1_ref[...], b1_ref[...], cnt)
    gp = g_ref[:, 64:80]
    p = _pcd16(p_ref)
    prep = jnp.reshape(
        jnp.broadcast_to(p[:, None, :], (pb, N_NEI, 16)), (rb, 16))
    pr = prep - gp
    pe1 = jax.lax.dot_general(
        pr, w1_ref[...], (((1,), (1,)), ((), ())),
        preferred_element_type=jnp.float32) + b1_ref[...]
    xn = (pe1 - mean1) * jax.lax.rsqrt(var1 + EPS) * g1_ref[...] + be1_ref[...]
    z = jnp.maximum(xn, 0.0)
    return jax.lax.dot_general(
        z, w2_ref[...], (((1,), (1,)), ((), ())),
        preferred_element_type=jnp.float32) + b2_ref[...]


def _fused_body(g_ref, p_ref, f_ref,
                pw1_ref, pb1_ref, pg1_ref, pbe1_ref, pw2_ref, pb2_ref,
                aw1_ref, ab1_ref, ag1_ref, abe1_ref, aw2_ref, ab2_ref,
                out_ref, acc1_ref, acc2_ref, *, nblk, cnt):
    i = pl.program_id(0)
    ph = i // nblk
    pb = p_ref.shape[2]
    rb = pb * N_NEI

    @pl.when(i == 0)
    def _():
        acc1_ref[...] = jnp.zeros_like(acc1_ref)

    @pl.when(i == nblk)
    def _():
        acc2_ref[...] = jnp.zeros_like(acc2_ref)

    @pl.when(ph == 0)
    def _():
        gp = g_ref[:, 64:80]
        p = _pcd16(p_ref)
        prep = jnp.reshape(
            jnp.broadcast_to(p[:, None, :], (pb, N_NEI, 16)), (rb, 16))
        pr = prep - gp
        outer = jax.lax.dot_general(
            pr, pr, (((0,), (0,)), ((), ())),
            preferred_element_type=jnp.float32)
        acc1_ref[0:16, :] += outer
        acc1_ref[16:17, :] += jnp.sum(pr, axis=0)[None, :]

    @pl.when(ph == 1)
    def _():
        pe = _compute_pe(g_ref, p_ref, acc1_ref, pw1_ref, pb1_ref, pg1_ref,
                         pbe1_ref, pw2_ref, pb2_ref, cnt)
        f = jnp.transpose(f_ref[0])
        frep = jnp.reshape(
            jnp.broadcast_to(f[:, None, :], (pb, N_NEI, 64)), (rb, 64))
        x2 = (frep - g_ref[:, 0:64]) + pe
        outer2 = jax.lax.dot_general(
            x2, x2, (((0,), (0,)), ((), ())),
            preferred_element_type=jnp.float32)
        acc2_ref[0:64, :] += outer2
        acc2_ref[64:65, :] += jnp.sum(x2, axis=0)[None, :]

    @pl.when(ph == 2)
    def _():
        hid = aw1_ref.shape[0]
        pe = _compute_pe(g_ref, p_ref, acc1_ref, pw1_ref, pb1_ref, pg1_ref,
                         pbe1_ref, pw2_ref, pb2_ref, cnt)
        mean2, var2 = _bn_stats(acc2_ref[...], 64, aw1_ref[...],
                                ab1_ref[...], cnt)
        f = jnp.transpose(f_ref[0])
        frep = jnp.reshape(
            jnp.broadcast_to(f[:, None, :], (pb, N_NEI, 64)), (rb, 64))
        gfeat = g_ref[:, 0:64]
        x2 = (frep - gfeat) + pe
        ap = jax.lax.dot_general(
            x2, aw1_ref[...], (((1,), (1,)), ((), ())),
            preferred_element_type=jnp.float32) + ab1_ref[...]
        an = ((ap - mean2) * jax.lax.rsqrt(var2 + EPS) * ag1_ref[...]
              + abe1_ref[...])
        an = jnp.maximum(an, 0.0)
        wp = jax.lax.dot_general(
            an, aw2_ref[...], (((1,), (1,)), ((), ())),
            preferred_element_type=jnp.float32) + ab2_ref[...]
        wp3 = jnp.reshape(wp, (pb, N_NEI, 64))
        m = jnp.max(wp3, axis=1, keepdims=True)
        e = jnp.exp(wp3 - m)
        sm = e / jnp.sum(e, axis=1, keepdims=True)
        gf3 = jnp.reshape(gfeat + pe, (pb, N_NEI, 64))
        out = jnp.sum(sm * gf3, axis=1)
        out_ref[0] = jnp.transpose(out)


def _fused(g, pcd, feat, pw1, pb1, pg1, pbe1, pw2, pb2,
           aw1, ab1, ag1, abe1, aw2, ab2, rb):
    rows = g.shape[0]
    nblk = rows // rb
    pb = rb // N_NEI
    B, C, N = feat.shape
    npb = N // pb
    hid = aw1.shape[0]
    cnt = float(rows)

    def gmap(i):
        return (i % nblk, 0)

    def pmap(i):
        return ((i % nblk) // npb, 0, (i % nblk) % npb)

    def cmap(i):
        return (0, 0)

    out, _, _ = pl.pallas_call(
        functools.partial(_fused_body, nblk=nblk, cnt=cnt),
        grid=(3 * nblk,),
        in_specs=[
            pl.BlockSpec((rb, D_TAB), gmap),
            pl.BlockSpec((1, 3, pb), pmap),
            pl.BlockSpec((1, 64, pb), pmap),
            pl.BlockSpec((64, 16), cmap),
            pl.BlockSpec((1, 64), cmap),
            pl.BlockSpec((1, 64), cmap),
            pl.BlockSpec((1, 64), cmap),
            pl.BlockSpec((64, 64), cmap),
            pl.BlockSpec((1, 64), cmap),
            pl.BlockSpec((hid, 64), cmap),
            pl.BlockSpec((1, hid), cmap),
            pl.BlockSpec((1, hid), cmap),
            pl.BlockSpec((1, hid), cmap),
            pl.BlockSpec((64, hid), cmap),
            pl.BlockSpec((1, 64), cmap),
        ],
        out_specs=[
            pl.BlockSpec((1, C, pb), pmap),
            pl.BlockSpec((24, 16), cmap),
            pl.BlockSpec((72, 64), cmap),
        ],
        out_shape=[
            jax.ShapeDtypeStruct((B, C, N), jnp.float32),
            jax.ShapeDtypeStruct((24, 16), jnp.float32),
            jax.ShapeDtypeStruct((72, 64), jnp.float32),
        ],
    )(g, pcd, feat, pw1, pb1, pg1, pbe1, pw2, pb2,
      aw1, ab1, ag1, abe1, aw2, ab2)
    return out


def kernel(pcd, feat, pcd_feadb, feat_feadb,
           pos_w1, pos_b1, pos_g1, pos_be1, pos_w2, pos_b2,
           attn_w1, attn_b1, attn_g1, attn_be1, attn_w2, attn_b2):
    B, C, N = feat.shape
    rows = B * N * N_NEI
    RB = 4096

    table, rsq = _table(feat, feat_feadb, pcd, pcd_feadb)
    idx = _knn(feat, table, rsq)
    g = _sc_gather(table, idx.reshape(rows))

    w1p = jnp.concatenate(
        [pos_w1, jnp.zeros((pos_w1.shape[0], 13), jnp.float32)], axis=1)
    return _fused(g, pcd, feat, w1p,
                  pos_b1[None, :], pos_g1[None, :], pos_be1[None, :],
                  pos_w2, pos_b2[None, :],
                  attn_w1, attn_b1[None, :], attn_g1[None, :],
                  attn_be1[None, :], attn_w2, attn_b2[None, :], RB)
